# Initial kernel scaffold; baseline (speedup 1.0000x reference)
import jax
import jax.numpy as jnp
from jax.experimental import pallas as pl


def kernel(x, edge_index, W_in, b_in, Wl1, Wr1, bl1, br1, att1, bias1, Wl2, Wr2, bl2, br2, att2, bias2, W_out, b_out):
    raise NotImplementedError("write your pallas kernel here")


# speedup vs baseline: 23.3269x; 23.3269x over previous
import functools

import jax
import jax.numpy as jnp
from jax import lax
from jax.experimental import pallas as pl
from jax.experimental.pallas import tpu as pltpu
from jax.experimental.pallas import tpu_sc as plsc

N = 10000
E = 320000
D_IN = 128
H1 = 128
C2 = 32
D_OUT = 96

NC = 2
NS = 16
NW = NC * NS
EPT = E // NW
ROW_BLK = 1000
GRID = N // ROW_BLK


def _lrelu(s):
    return jnp.maximum(s, 0.2 * s)


def _lane_sum(v):
    dn = lax.GatherDimensionNumbers(
        offset_dims=(), collapsed_slice_dims=(0,), start_index_map=(0,))
    lane = lax.iota(jnp.int32, 16)
    for k in (8, 4, 2, 1):
        idx = lax.bitwise_xor(lane, k)
        shuf = lax.gather(v, idx[:, None], dn, (1,),
                          mode=lax.GatherScatterMode.PROMISE_IN_BOUNDS)
        v = v + shuf
    return v


def _proj_body(x_ref, wi_ref, bi_ref, wl_ref, wr_ref, bl_ref, br_ref,
               xl_ref, xr_ref):
    h = jnp.dot(x_ref[...], wi_ref[...],
                preferred_element_type=jnp.float32) + bi_ref[...]
    xl_ref[...] = jnp.dot(h, wl_ref[...],
                          preferred_element_type=jnp.float32) + bl_ref[...]
    xr_ref[...] = jnp.dot(h, wr_ref[...],
                          preferred_element_type=jnp.float32) + br_ref[...]


def _proj(x, W_in, b_in, Wl1, Wr1, bl1, br1):
    full = lambda s: pl.BlockSpec(s, lambda i: (0, 0))
    return pl.pallas_call(
        _proj_body,
        grid=(GRID,),
        in_specs=[
            pl.BlockSpec((ROW_BLK, D_IN), lambda i: (i, 0)),
            full((D_IN, D_IN)), full((1, D_IN)),
            full((D_IN, H1)), full((D_IN, H1)),
            full((1, H1)), full((1, H1)),
        ],
        out_specs=[
            pl.BlockSpec((ROW_BLK, H1), lambda i: (i, 0)),
            pl.BlockSpec((ROW_BLK, H1), lambda i: (i, 0)),
        ],
        out_shape=[
            jax.ShapeDtypeStruct((N, H1), jnp.float32),
            jax.ShapeDtypeStruct((N, H1), jnp.float32),
        ],
    )(x, W_in, b_in.reshape(1, -1), Wl1, Wr1,
      bl1.reshape(1, -1), br1.reshape(1, -1))


NPAD = 10240
RPT = NPAD // NS


def _make_edge_kernel(vw, batch, head_split):
    nslc = vw // 16
    ept = E // NS if head_split else E // NW
    chunks = ept // batch
    mesh = plsc.VectorSubcoreMesh(core_axis_name="c", subcore_axis_name="s")

    @functools.partial(
        pl.kernel,
        out_type=jax.ShapeDtypeStruct((NC, NPAD, 128), jnp.float32),
        mesh=mesh,
        compiler_params=pltpu.CompilerParams(use_tc_tiling_on_sc=False),
        scratch_types=[
            pltpu.VMEM((batch,), jnp.int32),
            pltpu.VMEM((batch,), jnp.int32),
            pltpu.VMEM((batch,), jnp.int32),
            pltpu.VMEM((batch,), jnp.int32),
            pltpu.VMEM((batch, vw), jnp.float32),
            pltpu.VMEM((batch, vw), jnp.float32),
            pltpu.VMEM((batch, 128), jnp.float32),
            pltpu.VMEM((vw,), jnp.float32),
            pltpu.VMEM_SHARED((NPAD, 128), jnp.float32),
            pltpu.SemaphoreType.DMA,
            pltpu.SemaphoreType.DMA,
        ],
    )
    def edge_kernel(src_hbm, dst_hbm, xl_hbm, xr_hbm, att_hbm, zer_hbm,
                    out_hbm, srcv, dstv, gsv, gdv, xlv, xrv, valv, attv,
                    acc, sem1, sem2):
        c = lax.axis_index("c")
        s = lax.axis_index("s")
        pltpu.sync_copy(zer_hbm, acc.at[pl.ds(s * RPT, RPT)])
        if head_split:
            pltpu.sync_copy(att_hbm.at[pl.ds(c * vw, vw)], attv)
            base0 = s * ept
        else:
            pltpu.sync_copy(att_hbm, attv)
            base0 = (s * NC + c) * ept
        plsc.subcore_barrier()

        lane = lax.iota(jnp.int32, 16)
        mask0 = jnp.where(lane == 0, 1.0, 0.0)
        zeros16 = jnp.zeros((16,), jnp.float32)

        def tail_body(e, carry):
            for j in range(nslc + 1, 8):
                valv[e, pl.ds(j * 16, 16)] = zeros16
            return carry
        lax.fori_loop(0, batch, tail_body, 0)

        atts = [attv[pl.ds(j * 16, 16)] for j in range(nslc)]

        def chunk_body(k, carry):
            base = base0 + k * batch
            pltpu.sync_copy(src_hbm.at[pl.ds(base, batch)], srcv)
            pltpu.sync_copy(dst_hbm.at[pl.ds(base, batch)], dstv)
            if head_split:
                def idx_body(g, carry2):
                    off = pl.ds(g * 16, 16)
                    gsv[off] = srcv[off] * 2 + c
                    gdv[off] = dstv[off] * 2 + c
                    return carry2
                lax.fori_loop(0, batch // 16, idx_body, 0)
                pltpu.async_copy(xl_hbm.at[gsv], xlv, sem1).wait()
                pltpu.async_copy(xr_hbm.at[gdv], xrv, sem2).wait()
            else:
                pltpu.async_copy(xl_hbm.at[srcv], xlv, sem1).wait()
                pltpu.async_copy(xr_hbm.at[dstv], xrv, sem2).wait()

            def edge_body(e, carry2):
                xs = []
                lacc = zeros16
                for j in range(nslc):
                    a = xlv[e, pl.ds(j * 16, 16)]
                    b = xrv[e, pl.ds(j * 16, 16)]
                    lacc = lacc + _lrelu(a + b) * atts[j]
                    xs.append(a)
                p = jnp.exp(_lane_sum(lacc))
                for j in range(nslc):
                    valv[e, pl.ds(j * 16, 16)] = xs[j] * p
                valv[e, pl.ds(vw, 16)] = p * mask0
                return carry2

            lax.fori_loop(0, batch, edge_body, 0)
            pltpu.sync_copy(valv, acc.at[dstv], add=True)
            return carry

        lax.fori_loop(0, chunks, chunk_body, 0)
        plsc.subcore_barrier()
        pltpu.sync_copy(acc.at[pl.ds(s * RPT, RPT)],
                        out_hbm.at[c, pl.ds(s * RPT, RPT)])

    return edge_kernel


_edge1 = _make_edge_kernel(vw=64, batch=80, head_split=True)
_edge2 = _make_edge_kernel(vw=C2, batch=80, head_split=False)


def _mid_body(xl_ref, xr_ref, pa_ref, pb_ref, att_ref, b1_ref,
              wl_ref, wr_ref, bl_ref, br_ref, xl2_ref, xr2_ref):
    xl = xl_ref[...]
    pa = pa_ref[...]
    pb = pb_ref[...]
    t = _lrelu(xl + xr_ref[...]) * att_ref[...]
    p0 = jnp.exp(jnp.sum(t[:, 0:64], axis=1, keepdims=True))
    p1 = jnp.exp(jnp.sum(t[:, 64:128], axis=1, keepdims=True))
    den0 = pa[:, 64:65] + p0
    den1 = pb[:, 64:65] + p1
    num0 = pa[:, 0:64] + p0 * xl[:, 0:64]
    num1 = pb[:, 0:64] + p1 * xl[:, 64:128]
    h1 = jnp.concatenate([num0 / den0, num1 / den1], axis=1) + b1_ref[...]
    h1 = jnp.maximum(h1, 0.0)
    xl2_ref[...] = jnp.dot(h1, wl_ref[...],
                           preferred_element_type=jnp.float32) + bl_ref[...]
    xr2_ref[...] = jnp.dot(h1, wr_ref[...],
                           preferred_element_type=jnp.float32) + br_ref[...]


def _mid(xl1, xr1, part1, att1, bias1, Wl2, Wr2, bl2, br2):
    full = lambda s: pl.BlockSpec(s, lambda i: (0, 0))
    return pl.pallas_call(
        _mid_body,
        grid=(GRID,),
        in_specs=[
            pl.BlockSpec((ROW_BLK, H1), lambda i: (i, 0)),
            pl.BlockSpec((ROW_BLK, H1), lambda i: (i, 0)),
            pl.BlockSpec((ROW_BLK, 128), lambda i: (i, 0)),
            pl.BlockSpec((ROW_BLK, 128), lambda i: (i, 0)),
            full((1, H1)), full((1, H1)),
            full((H1, C2)), full((H1, C2)),
            full((1, C2)), full((1, C2)),
        ],
        out_specs=[
            pl.BlockSpec((ROW_BLK, C2), lambda i: (i, 0)),
            pl.BlockSpec((ROW_BLK, C2), lambda i: (i, 0)),
        ],
        out_shape=[
            jax.ShapeDtypeStruct((N, C2), jnp.float32),
            jax.ShapeDtypeStruct((N, C2), jnp.float32),
        ],
    )(xl1, xr1, part1[0], part1[1], att1.reshape(1, -1),
      bias1.reshape(1, -1), Wl2, Wr2, bl2.reshape(1, -1), br2.reshape(1, -1))


def _out_body(xl_ref, xr_ref, pa_ref, pb_ref, att_ref, b2_ref,
              wo_ref, bo_ref, o_ref):
    xl = xl_ref[...]
    raw = pa_ref[...] + pb_ref[...]
    t = _lrelu(xl + xr_ref[...]) * att_ref[...]
    p = jnp.exp(jnp.sum(t, axis=1, keepdims=True))
    den = raw[:, C2:C2 + 1] + p
    num = raw[:, 0:C2] + p * xl
    h2 = jnp.maximum(num / den + b2_ref[...], 0.0)
    g = jnp.sum(h2, axis=0, keepdims=True) * (1.0 / N)
    o_ref[...] = jnp.dot(g, wo_ref[...],
                         preferred_element_type=jnp.float32) + bo_ref[...]


def _out(xl2, xr2, part2, att2, bias2, W_out, b_out):
    return pl.pallas_call(
        _out_body,
        grid=(1,),
        in_specs=[
            pl.BlockSpec((N, C2), lambda i: (0, 0)),
            pl.BlockSpec((N, C2), lambda i: (0, 0)),
            pl.BlockSpec((N, 128), lambda i: (0, 0)),
            pl.BlockSpec((N, 128), lambda i: (0, 0)),
            pl.BlockSpec((1, C2), lambda i: (0, 0)),
            pl.BlockSpec((1, C2), lambda i: (0, 0)),
            pl.BlockSpec((C2, D_OUT), lambda i: (0, 0)),
            pl.BlockSpec((1, D_OUT), lambda i: (0, 0)),
        ],
        out_specs=pl.BlockSpec((1, D_OUT), lambda i: (0, 0)),
        out_shape=jax.ShapeDtypeStruct((1, D_OUT), jnp.float32),
    )(xl2, xr2, part2[0], part2[1], att2.reshape(1, -1),
      bias2.reshape(1, -1), W_out, b_out.reshape(1, -1))


def kernel(x, edge_index, W_in, b_in, Wl1, Wr1, bl1, br1, att1, bias1,
           Wl2, Wr2, bl2, br2, att2, bias2, W_out, b_out):
    src = edge_index[0]
    dst = edge_index[1]
    zer = jnp.zeros((RPT, 128), jnp.float32)

    xl1, xr1 = _proj(x, W_in, b_in, Wl1, Wr1, bl1, br1)
    part1 = _edge1(src, dst, xl1.reshape(2 * N, 64), xr1.reshape(2 * N, 64),
                   att1.reshape(-1), zer)
    xl2, xr2 = _mid(xl1, xr1, part1, att1, bias1, Wl2, Wr2, bl2, br2)
    part2 = _edge2(src, dst, xl2, xr2, att2.reshape(-1), zer)
    return _out(xl2, xr2, part2, att2, bias2, W_out, b_out)


# speedup vs baseline: 46.0190x; 1.9728x over previous
import functools

import jax
import jax.numpy as jnp
from jax import lax
from jax.experimental import pallas as pl
from jax.experimental.pallas import tpu as pltpu
from jax.experimental.pallas import tpu_sc as plsc

N = 10000
E = 320000
D_IN = 128
H1 = 128
C2 = 32
D_OUT = 96

NC = 2
NS = 16
NW = NC * NS
EPT = E // NW
ROW_BLK = 1000
GRID = N // ROW_BLK


def _lrelu(s):
    return jnp.maximum(s, 0.2 * s)


def _lane_sum(v):
    dn = lax.GatherDimensionNumbers(
        offset_dims=(), collapsed_slice_dims=(0,), start_index_map=(0,))
    lane = lax.iota(jnp.int32, 16)
    for k in (8, 4, 2, 1):
        idx = lax.bitwise_xor(lane, k)
        shuf = lax.gather(v, idx[:, None], dn, (1,),
                          mode=lax.GatherScatterMode.PROMISE_IN_BOUNDS)
        v = v + shuf
    return v


def _proj_body(x_ref, wi_ref, bi_ref, wl_ref, wr_ref, bl_ref, br_ref,
               xl_ref, xr_ref):
    h = jnp.dot(x_ref[...], wi_ref[...],
                preferred_element_type=jnp.float32) + bi_ref[...]
    xl_ref[...] = jnp.dot(h, wl_ref[...],
                          preferred_element_type=jnp.float32) + bl_ref[...]
    xr_ref[...] = jnp.dot(h, wr_ref[...],
                          preferred_element_type=jnp.float32) + br_ref[...]


def _proj(x, W_in, b_in, Wl1, Wr1, bl1, br1):
    full = lambda s: pl.BlockSpec(s, lambda i: (0, 0))
    return pl.pallas_call(
        _proj_body,
        grid=(GRID,),
        in_specs=[
            pl.BlockSpec((ROW_BLK, D_IN), lambda i: (i, 0)),
            full((D_IN, D_IN)), full((1, D_IN)),
            full((D_IN, H1)), full((D_IN, H1)),
            full((1, H1)), full((1, H1)),
        ],
        out_specs=[
            pl.BlockSpec((ROW_BLK, H1), lambda i: (i, 0)),
            pl.BlockSpec((ROW_BLK, H1), lambda i: (i, 0)),
        ],
        out_shape=[
            jax.ShapeDtypeStruct((N, H1), jnp.float32),
            jax.ShapeDtypeStruct((N, H1), jnp.float32),
        ],
    )(x, W_in, b_in.reshape(1, -1), Wl1, Wr1,
      bl1.reshape(1, -1), br1.reshape(1, -1))


NPAD = 10240
RPT = NPAD // NS


def _make_edge_kernel(vw, batch, head_split):
    nslc = vw // 16
    nwork = NS if head_split else NW
    ept = E // nwork
    chunks = ept // batch
    mesh = plsc.VectorSubcoreMesh(core_axis_name="c", subcore_axis_name="s")

    @functools.partial(
        pl.kernel,
        out_type=jax.ShapeDtypeStruct((NC, NPAD, 128), jnp.float32),
        mesh=mesh,
        compiler_params=pltpu.CompilerParams(use_tc_tiling_on_sc=False),
        scratch_types=[
            pltpu.VMEM((2, batch), jnp.int32),
            pltpu.VMEM((2, batch), jnp.int32),
            pltpu.VMEM((2, batch), jnp.int32),
            pltpu.VMEM((2, batch), jnp.int32),
            pltpu.VMEM((4, batch), jnp.int32),
            pltpu.VMEM((batch, vw), jnp.float32),
            pltpu.VMEM((batch, vw), jnp.float32),
            pltpu.VMEM((batch, vw), jnp.float32),
            pltpu.VMEM((batch, vw), jnp.float32),
            pltpu.VMEM((batch, 128), jnp.float32),
            pltpu.VMEM((batch, 128), jnp.float32),
            pltpu.VMEM((vw,), jnp.float32),
            pltpu.VMEM_SHARED((NPAD, 128), jnp.float32),
            pltpu.SemaphoreType.DMA, pltpu.SemaphoreType.DMA,
            pltpu.SemaphoreType.DMA, pltpu.SemaphoreType.DMA,
            pltpu.SemaphoreType.DMA, pltpu.SemaphoreType.DMA,
            pltpu.SemaphoreType.DMA, pltpu.SemaphoreType.DMA,
        ],
    )
    def edge_kernel(src_hbm, dst_hbm, xl_hbm, xr_hbm, att_hbm, zer_hbm,
                    out_hbm, ibs, ibd, gs, gd, dsc, xl0, xl1, xr0, xr1,
                    val0, val1, attv, acc, sis0, sis1, sid0, sid1,
                    sxl0, sxl1, ssc0, ssc1):
        c = lax.axis_index("c")
        s = lax.axis_index("s")
        xlv = [xl0, xl1]
        xrv = [xr0, xr1]
        valv = [val0, val1]
        sidx = [(sis0, sid0), (sis1, sid1)]
        sxl = [sxl0, sxl1]
        ssc = [ssc0, ssc1]

        pltpu.sync_copy(zer_hbm, acc.at[pl.ds(s * RPT, RPT)])
        if head_split:
            pltpu.sync_copy(att_hbm.at[pl.ds(c * vw, vw)], attv)
            widx = s
        else:
            pltpu.sync_copy(att_hbm, attv)
            widx = s * NC + c
        plsc.subcore_barrier()

        lane = lax.iota(jnp.int32, 16)
        mask0 = jnp.where(lane == 0, 1.0, 0.0)
        zeros16 = jnp.zeros((16,), jnp.float32)

        def tail_body(e, carry):
            for j in range(nslc + 1, 8):
                val0[e, pl.ds(j * 16, 16)] = zeros16
                val1[e, pl.ds(j * 16, 16)] = zeros16
            return carry
        lax.fori_loop(0, batch, tail_body, 0)

        atts = [attv[pl.ds(j * 16, 16)] for j in range(nslc)]

        def issue_idx(k, b):
            pltpu.async_copy(src_hbm.at[widx, k], ibs.at[b], sidx[b][0])
            pltpu.async_copy(dst_hbm.at[widx, k], ibd.at[b], sidx[b][1])

        def wait_idx(b):
            pltpu.make_async_copy(src_hbm.at[widx, 0], ibs.at[b],
                                  sidx[b][0]).wait()
            pltpu.make_async_copy(dst_hbm.at[widx, 0], ibd.at[b],
                                  sidx[b][1]).wait()

        def transform_idx(b, slot):
            def idx_body(g, carry2):
                off = pl.ds(g * 16, 16)
                if head_split:
                    gs[b, off] = ibs[b, off] * 2 + c
                    gd[b, off] = ibd[b, off] * 2 + c
                else:
                    gs[b, off] = ibs[b, off]
                    gd[b, off] = ibd[b, off]
                dsc[slot, off] = ibd[b, off]
                return carry2
            lax.fori_loop(0, batch // 16, idx_body, 0)

        def issue_gathers(b):
            pltpu.async_copy(xl_hbm.at[gs.at[b]], xlv[b], sxl[b])
            pltpu.async_copy(xr_hbm.at[gd.at[b]], xrv[b], sxl[b])

        def wait_gathers(b):
            pltpu.make_async_copy(xl_hbm.at[gs.at[b]], xlv[b], sxl[b]).wait()
            pltpu.make_async_copy(xr_hbm.at[gd.at[b]], xrv[b], sxl[b]).wait()

        def wait_scatter(b):
            pltpu.make_async_copy(valv[b], acc.at[dsc.at[0]], ssc[b]).wait()

        def compute_chunk(xlb, xrb, valb):
            def edge_body(e, carry2):
                xs = []
                lacc = zeros16
                for j in range(nslc):
                    a = xlb[e, pl.ds(j * 16, 16)]
                    r = xrb[e, pl.ds(j * 16, 16)]
                    lacc = lacc + _lrelu(a + r) * atts[j]
                    xs.append(a)
                p = jnp.exp(_lane_sum(lacc))
                for j in range(nslc):
                    valb[e, pl.ds(j * 16, 16)] = xs[j] * p
                valb[e, pl.ds(vw, 16)] = p * mask0
                return carry2
            lax.fori_loop(0, batch, edge_body, 0)

        def _when(cond, fn):
            if isinstance(cond, bool):
                if cond:
                    fn()
            else:
                pl.when(cond)(fn)

        def iteration(k, i, first=False):
            b = i % 2
            bn = 1 - b
            scur = i
            snxt = (i + 1) % 4
            wait_gathers(b)
            def _stage_next():
                wait_idx(bn)
                transform_idx(bn, snxt)
                issue_gathers(bn)

            def _fetch_idx():
                issue_idx(k + 3, bn)

            _when(k + 1 < chunks, _stage_next)
            _when(k + 3 < chunks, _fetch_idx)
            if not first:
                wait_scatter(b)
            compute_chunk(xlv[b], xrv[b], valv[b])
            pltpu.async_copy(valv[b], acc.at[dsc.at[scur]], ssc[b], add=True)

        issue_idx(0, 0)
        wait_idx(0)
        transform_idx(0, 0)
        issue_gathers(0)
        issue_idx(1, 1)
        issue_idx(2, 0)
        iteration(0, 0, first=True)
        iteration(1, 1, first=True)
        iteration(2, 2)
        iteration(3, 3)

        def quad_body(m, carry):
            for i in range(4):
                iteration(4 * m + i, i)
            return carry
        lax.fori_loop(1, chunks // 4, quad_body, 0)
        for k in range(4 * (chunks // 4), chunks):
            iteration(k, k % 4)
        wait_scatter(0)
        wait_scatter(1)

        plsc.subcore_barrier()
        pltpu.sync_copy(acc.at[pl.ds(s * RPT, RPT)],
                        out_hbm.at[c, pl.ds(s * RPT, RPT)])

    return edge_kernel


_edge1 = _make_edge_kernel(vw=64, batch=80, head_split=True)
_edge2 = _make_edge_kernel(vw=C2, batch=80, head_split=False)


def _mid_body(xl_ref, xr_ref, pa_ref, pb_ref, att_ref, b1_ref,
              wl_ref, wr_ref, bl_ref, br_ref, xl2_ref, xr2_ref):
    xl = xl_ref[...]
    pa = pa_ref[...]
    pb = pb_ref[...]
    t = _lrelu(xl + xr_ref[...]) * att_ref[...]
    p0 = jnp.exp(jnp.sum(t[:, 0:64], axis=1, keepdims=True))
    p1 = jnp.exp(jnp.sum(t[:, 64:128], axis=1, keepdims=True))
    den0 = pa[:, 64:65] + p0
    den1 = pb[:, 64:65] + p1
    num0 = pa[:, 0:64] + p0 * xl[:, 0:64]
    num1 = pb[:, 0:64] + p1 * xl[:, 64:128]
    h1 = jnp.concatenate([num0 / den0, num1 / den1], axis=1) + b1_ref[...]
    h1 = jnp.maximum(h1, 0.0)
    xl2_ref[...] = jnp.dot(h1, wl_ref[...],
                           preferred_element_type=jnp.float32) + bl_ref[...]
    xr2_ref[...] = jnp.dot(h1, wr_ref[...],
                           preferred_element_type=jnp.float32) + br_ref[...]


def _mid(xl1, xr1, part1, att1, bias1, Wl2, Wr2, bl2, br2):
    full = lambda s: pl.BlockSpec(s, lambda i: (0, 0))
    return pl.pallas_call(
        _mid_body,
        grid=(GRID,),
        in_specs=[
            pl.BlockSpec((ROW_BLK, H1), lambda i: (i, 0)),
            pl.BlockSpec((ROW_BLK, H1), lambda i: (i, 0)),
            pl.BlockSpec((ROW_BLK, 128), lambda i: (i, 0)),
            pl.BlockSpec((ROW_BLK, 128), lambda i: (i, 0)),
            full((1, H1)), full((1, H1)),
            full((H1, C2)), full((H1, C2)),
            full((1, C2)), full((1, C2)),
        ],
        out_specs=[
            pl.BlockSpec((ROW_BLK, C2), lambda i: (i, 0)),
            pl.BlockSpec((ROW_BLK, C2), lambda i: (i, 0)),
        ],
        out_shape=[
            jax.ShapeDtypeStruct((N, C2), jnp.float32),
            jax.ShapeDtypeStruct((N, C2), jnp.float32),
        ],
    )(xl1, xr1, part1[0], part1[1], att1.reshape(1, -1),
      bias1.reshape(1, -1), Wl2, Wr2, bl2.reshape(1, -1), br2.reshape(1, -1))


def _out_body(xl_ref, xr_ref, pa_ref, pb_ref, att_ref, b2_ref,
              wo_ref, bo_ref, o_ref):
    xl = xl_ref[...]
    raw = pa_ref[...] + pb_ref[...]
    t = _lrelu(xl + xr_ref[...]) * att_ref[...]
    p = jnp.exp(jnp.sum(t, axis=1, keepdims=True))
    den = raw[:, C2:C2 + 1] + p
    num = raw[:, 0:C2] + p * xl
    h2 = jnp.maximum(num / den + b2_ref[...], 0.0)
    g = jnp.sum(h2, axis=0, keepdims=True) * (1.0 / N)
    o_ref[...] = jnp.dot(g, wo_ref[...],
                         preferred_element_type=jnp.float32) + bo_ref[...]


def _out(xl2, xr2, part2, att2, bias2, W_out, b_out):
    return pl.pallas_call(
        _out_body,
        grid=(1,),
        in_specs=[
            pl.BlockSpec((N, C2), lambda i: (0, 0)),
            pl.BlockSpec((N, C2), lambda i: (0, 0)),
            pl.BlockSpec((N, 128), lambda i: (0, 0)),
            pl.BlockSpec((N, 128), lambda i: (0, 0)),
            pl.BlockSpec((1, C2), lambda i: (0, 0)),
            pl.BlockSpec((1, C2), lambda i: (0, 0)),
            pl.BlockSpec((C2, D_OUT), lambda i: (0, 0)),
            pl.BlockSpec((1, D_OUT), lambda i: (0, 0)),
        ],
        out_specs=pl.BlockSpec((1, D_OUT), lambda i: (0, 0)),
        out_shape=jax.ShapeDtypeStruct((1, D_OUT), jnp.float32),
    )(xl2, xr2, part2[0], part2[1], att2.reshape(1, -1),
      bias2.reshape(1, -1), W_out, b_out.reshape(1, -1))


def kernel(x, edge_index, W_in, b_in, Wl1, Wr1, bl1, br1, att1, bias1,
           Wl2, Wr2, bl2, br2, att2, bias2, W_out, b_out):
    src = edge_index[0]
    dst = edge_index[1]
    zer = jnp.zeros((RPT, 128), jnp.float32)

    src1 = src.reshape(NS, E // NS // 80, 80)
    dst1 = dst.reshape(NS, E // NS // 80, 80)
    src2 = src.reshape(NW, E // NW // 80, 80)
    dst2 = dst.reshape(NW, E // NW // 80, 80)

    xl1, xr1 = _proj(x, W_in, b_in, Wl1, Wr1, bl1, br1)
    part1 = _edge1(src1, dst1, xl1.reshape(2 * N, 64), xr1.reshape(2 * N, 64),
                   att1.reshape(-1), zer)
    xl2, xr2 = _mid(xl1, xr1, part1, att1, bias1, Wl2, Wr2, bl2, br2)
    part2 = _edge2(src2, dst2, xl2, xr2, att2.reshape(-1), zer)
    return _out(xl2, xr2, part2, att2, bias2, W_out, b_out)
